# baseline (speedup 1.0000x reference)
import functools

import jax
import jax.numpy as jnp
from jax import lax
from jax.experimental import pallas as pl
from jax.experimental.pallas import tpu as pltpu
from jax.experimental.pallas import tpu_sc as plsc

F = 26
D = 16
B = 16384
FIELD_SIZE = 38462
V = F * FIELD_SIZE
VP = 1000064
NC, NS = 2, 16
NW = NC * NS
SPW = B // NW
BLK = 128
NBLK = SPW // BLK
ROWS = BLK * F

TBATCH = 1024
NFULL = V // TBATCH
TAIL0 = NFULL * TBATCH
TAIL1 = TAIL0 + 512
TAILC = V - TAIL1


def _idx_body(xt_ref, out_ref):
    for f in range(F):
        row = xt_ref[pl.ds(f, 1), :]
        out_ref[pl.ds(f * (B // 128), B // 128), :] = (
            jnp.reshape(row, (B // 128, 128)) + f * FIELD_SIZE)


@jax.jit
def _idx_call(x_t):
    return pl.pallas_call(
        _idx_body,
        out_shape=jax.ShapeDtypeStruct((F * B // 128, 128), jnp.int32),
    )(x_t)


def _detile_body(tbl_hbm, tail_hbm, out_hbm, stg, outv, sem):
    wid = lax.axis_index("s") * NC + lax.axis_index("c")
    iota = jax.lax.iota(jnp.int32, 16)

    def emit(base_col, ncol):
        def one_col(c, carry):
            outv[c, :] = plsc.load_gather(
                stg, [iota, jnp.broadcast_to(c, (16,)).astype(jnp.int32)])
            return carry
        lax.fori_loop(0, ncol, one_col, 0)
        pltpu.sync_copy(outv.at[pl.ds(0, ncol), :],
                        out_hbm.at[pl.ds(base_col, ncol), :])

    def do_batch(i, carry):
        b = wid * 31 + i
        pltpu.sync_copy(tbl_hbm.at[:, pl.ds(b * TBATCH, TBATCH)],
                        stg.at[:, pl.ds(0, TBATCH)])
        emit(b * TBATCH, TBATCH)
        return carry

    nb = jnp.where(wid < NW - 1, 31, NFULL - 31 * (NW - 1))
    lax.fori_loop(0, nb, do_batch, 0)

    @pl.when(wid == NW - 1)
    def _tail():
        pltpu.sync_copy(tbl_hbm.at[:, pl.ds(TAIL0, 512)],
                        stg.at[:, pl.ds(0, 512)])
        emit(TAIL0, 512)
        pltpu.sync_copy(tail_hbm, out_hbm.at[pl.ds(TAIL1, TAILC), :])


@jax.jit
def _detile_call(tbl_t, tail):
    k = pl.kernel(
        _detile_body,
        out_type=jax.ShapeDtypeStruct((VP, D), jnp.float32),
        mesh=plsc.VectorSubcoreMesh(core_axis_name="c", subcore_axis_name="s"),
        compiler_params=pltpu.CompilerParams(
            needs_layout_passes=False, use_tc_tiling_on_sc=False),
        scratch_types=[
            pltpu.VMEM((D, 1025), jnp.float32),
            pltpu.VMEM((TBATCH, D), jnp.float32),
            pltpu.SemaphoreType.DMA,
        ],
    )
    return k(tbl_t, tail)


def _fm_body(idx_hbm, table_hbm, out_hbm, idx_buf, rows_buf, out_buf, sem):
    wid = lax.axis_index("s") * NC + lax.axis_index("c")
    base = wid * SPW

    stages = []
    for f in range(F):
        stages.append(pltpu.async_copy(
            idx_hbm.at[pl.ds(f * (B // 128) + wid * NBLK, NBLK)],
            idx_buf.at[f], sem))
    for st in stages:
        st.wait()

    iota = jax.lax.iota(jnp.int32, 16)

    def do_block(j, carry):
        copies = []
        for f in range(F):
            copies.append(pltpu.async_copy(
                table_hbm.at[idx_buf.at[f, j]],
                rows_buf.at[pl.ds(f * BLK, BLK)], sem))
        for cp in copies:
            cp.wait()

        def do_group(g, carry2):
            def acc_field(f, accs):
                s_acc, q_acc = accs
                r0 = f * BLK + g * 16 + iota
                new_s = []
                new_q = []
                for d in range(D):
                    v = plsc.load_gather(rows_buf, [r0, (d + iota) % D])
                    new_s.append(s_acc[d] + v)
                    new_q.append(q_acc[d] + v * v)
                return (tuple(new_s), tuple(new_q))

            zero = jnp.zeros((16,), jnp.float32)
            init = (tuple(zero for _ in range(D)), tuple(zero for _ in range(D)))
            s_acc, q_acc = lax.fori_loop(0, F, acc_field, init)

            r = s_acc[0] * s_acc[0] - q_acc[0]
            for d in range(1, D):
                r = r + (s_acc[d] * s_acc[d] - q_acc[d])
            out_buf[pl.ds(j * BLK + g * 16, 16)] = 0.5 * r
            return carry2

        lax.fori_loop(0, BLK // 16, do_group, 0)
        return carry

    lax.fori_loop(0, NBLK, do_block, 0)

    pltpu.sync_copy(out_buf, out_hbm.at[pl.ds(base, SPW)])


@jax.jit
def _fm_call(idx, table_lin):
    k = pl.kernel(
        _fm_body,
        out_type=jax.ShapeDtypeStruct((B,), jnp.float32),
        mesh=plsc.VectorSubcoreMesh(core_axis_name="c", subcore_axis_name="s"),
        compiler_params=pltpu.CompilerParams(
            needs_layout_passes=False, use_tc_tiling_on_sc=False),
        scratch_types=[
            pltpu.VMEM((F, NBLK, BLK), jnp.int32),
            pltpu.VMEM((ROWS, D), jnp.float32),
            pltpu.VMEM((SPW,), jnp.float32),
            pltpu.SemaphoreType.DMA,
        ],
    )
    return k(idx, table_lin)


def kernel(x, table):
    idx = _idx_call(x.T)
    table_lin = _detile_call(table.T, table[TAIL1:])
    out = _fm_call(idx, table_lin)
    return out.reshape(B, 1)


# speedup vs baseline: 3.2884x; 3.2884x over previous
import functools

import jax
import jax.numpy as jnp
from jax import lax
from jax.experimental import pallas as pl
from jax.experimental.pallas import tpu as pltpu
from jax.experimental.pallas import tpu_sc as plsc

F = 26
D = 16
B = 16384
FIELD_SIZE = 38462
NC, NS = 2, 16
NW = NC * NS
SPW = B // NW
BLK = 128
NBLK = SPW // BLK
ROWS = BLK * F


def _idx_body(xt_ref, out_ref):
    for f in range(F):
        row = xt_ref[pl.ds(f, 1), :]
        out_ref[pl.ds(f * (B // 128), B // 128), :] = (
            jnp.reshape(row, (B // 128, 128)) + f * FIELD_SIZE)


@jax.jit
def _idx_call(x_t):
    return pl.pallas_call(
        _idx_body,
        out_shape=jax.ShapeDtypeStruct((F * B // 128, 128), jnp.int32),
    )(x_t)


def _fm_body(idx_hbm, table_hbm, out_hbm, idx_buf, rows_buf, out_buf, sem):
    wid = lax.axis_index("s") * NC + lax.axis_index("c")
    base = wid * SPW

    stages = []
    for f in range(F):
        stages.append(pltpu.async_copy(
            idx_hbm.at[pl.ds(f * (B // 128) + wid * NBLK, NBLK)],
            idx_buf.at[f], sem))
    for st in stages:
        st.wait()

    iota = jax.lax.iota(jnp.int32, 16)

    def fire(j, buf):
        return [pltpu.async_copy(
            table_hbm.at[idx_buf.at[f, j]],
            rows_buf.at[buf, pl.ds(f * BLK, BLK)], sem) for f in range(F)]

    def do_block(j, buf, copies_prev, copies_next):
        for cp in copies_prev:
            cp.wait()

        def do_group(g, carry2):
            def acc_field(f, accs):
                s_acc, q_acc = accs
                r0 = f * BLK + g * 16 + iota
                new_s = []
                new_q = []
                for d in range(D):
                    v = plsc.load_gather(
                        rows_buf, [jnp.broadcast_to(buf, (16,)).astype(jnp.int32), r0, (d + iota) % D])
                    new_s.append(s_acc[d] + v)
                    new_q.append(q_acc[d] + v * v)
                return (tuple(new_s), tuple(new_q))

            zero = jnp.zeros((16,), jnp.float32)
            init = (tuple(zero for _ in range(D)), tuple(zero for _ in range(D)))
            s_acc, q_acc = lax.fori_loop(0, F, acc_field, init)

            r = s_acc[0] * s_acc[0] - q_acc[0]
            for d in range(1, D):
                r = r + (s_acc[d] * s_acc[d] - q_acc[d])
            out_buf[pl.ds(j * BLK + g * 16, 16)] = 0.5 * r
            return carry2

        lax.fori_loop(0, BLK // 16, do_group, 0)

    c0 = fire(0, 0)
    c1 = fire(1, 1)
    do_block(0, 0, c0, None)
    c2 = fire(2, 0)
    do_block(1, 1, c1, None)
    c3 = fire(3, 1)
    do_block(2, 0, c2, None)
    do_block(3, 1, c3, None)

    pltpu.sync_copy(out_buf, out_hbm.at[pl.ds(base, SPW)])


@jax.jit
def _fm_call(idx, table):
    k = pl.kernel(
        _fm_body,
        out_type=jax.ShapeDtypeStruct((B,), jnp.float32),
        mesh=plsc.VectorSubcoreMesh(core_axis_name="c", subcore_axis_name="s"),
        compiler_params=pltpu.CompilerParams(
            needs_layout_passes=False, use_tc_tiling_on_sc=False),
        scratch_types=[
            pltpu.VMEM((F, NBLK, BLK), jnp.int32),
            pltpu.VMEM((2, ROWS, D), jnp.float32),
            pltpu.VMEM((SPW,), jnp.float32),
            pltpu.SemaphoreType.DMA,
        ],
    )
    return k(idx, table)


def kernel(x, table):
    idx = _idx_call(x.T)
    out = _fm_call(idx, table)
    return out.reshape(B, 1)
